# baseline (speedup 1.0000x reference)
import functools
import jax
import jax.numpy as jnp
from jax import lax
from jax.experimental import pallas as pl
from jax.experimental.pallas import tpu as pltpu
from jax.experimental.pallas import tpu_sc as plsc

D = 64
SCALE = 8.0
NC = 2
NS = 16
NW = NC * NS
LANES = 16
C = 128
NB = 4
FB = D // 8


def _make_kernel(L: int, NBB: int):
  units = L * NBB
  per_w = units // NW
  assert units % NW == 0 and per_w % NB == 0

  mesh = plsc.VectorSubcoreMesh(core_axis_name="c", subcore_axis_name="s")

  @functools.partial(
      pl.kernel,
      mesh=mesh,
      compiler_params=pltpu.CompilerParams(
          use_tc_tiling_on_sc=False, needs_layout_passes=False),
      out_type=jax.ShapeDtypeStruct((L, FB, NBB, 8, C), jnp.float32),
      scratch_types=[
          pltpu.VMEM((per_w, C), jnp.int32),
          pltpu.VMEM((NB, C, D), jnp.float32),
          pltpu.VMEM((NB, FB, 8, C), jnp.float32),
          [pltpu.SemaphoreType.DMA] * NB,
          [pltpu.SemaphoreType.DMA] * NB,
      ],
  )
  def body(x_hbm, table_hbm, out_hbm, idx_v, gbuf, tbuf, gsems, ssems):
    wid = lax.axis_index("s") * NC + lax.axis_index("c")
    ubase = wid * per_w
    pltpu.sync_copy(x_hbm.at[pl.ds(ubase, per_w)], idx_v)

    def start_gather(g, b):
      pltpu.async_copy(table_hbm.at[idx_v.at[g]], gbuf.at[b], gsems[b])

    def wait_gather(b):
      pltpu.make_async_copy(table_hbm.at[idx_v.at[0]], gbuf.at[b],
                            gsems[b]).wait()

    def start_store(g, b):
      u = ubase + g
      l = u // NBB
      bb = u - l * NBB
      pltpu.async_copy(tbuf.at[b], out_hbm.at[l, :, bb], ssems[b])

    def wait_store(b):
      pltpu.make_async_copy(tbuf.at[b], out_hbm.at[0, :, 0], ssems[b]).wait()

    for b in range(NB):
      start_gather(b, b)

    iotas = [lax.iota(jnp.int32, LANES) + (b0 * LANES) for b0 in range(8)]

    def unit(g, b):
      wait_gather(b)

      @pl.when(g >= NB)
      def _():
        wait_store(b)

      def frow(f, carry):
        cols = jnp.full((LANES,), f, jnp.int32)
        fb = f // 8
        fi = f - fb * 8
        for b0 in range(8):
          v = plsc.load_gather(gbuf.at[b], [iotas[b0], cols])
          tbuf[b, fb, fi, pl.ds(b0 * LANES, LANES)] = v * SCALE
        return carry

      lax.fori_loop(0, D, frow, 0)

      start_store(g, b)

      @pl.when(g + NB < per_w)
      def _():
        start_gather(g + NB, b)

    def outer(t, carry):
      for b in range(NB):
        unit(t + b, b)
      return carry

    lax.fori_loop(0, per_w // NB, lambda t, c: outer(t * NB, c), 0)

    for b in range(NB):
      wait_store(b)

  return body


def kernel(x, table):
  B, L = x.shape
  n = B * L
  nbb = B // C
  xu = x.astype(jnp.int32).T.reshape(L * nbb, C)
  out5 = _make_kernel(L, nbb)(xu, table)
  return out5.transpose(2, 4, 0, 1, 3).reshape(B, L, D)


# speedup vs baseline: 1.4791x; 1.4791x over previous
import functools
import jax
import jax.numpy as jnp
from jax import lax
from jax.experimental import pallas as pl
from jax.experimental.pallas import tpu as pltpu
from jax.experimental.pallas import tpu_sc as plsc

D = 64
SCALE = 8.0
NC = 2
NS = 16
NW = NC * NS
LANES = 16
C = 128
NB = 4
FB = D // 8


def _make_kernel(L: int, NBB: int):
  units = L * NBB
  per_w = units // NW
  assert units % NW == 0 and per_w % NB == 0

  mesh = plsc.VectorSubcoreMesh(core_axis_name="c", subcore_axis_name="s")

  @functools.partial(
      pl.kernel,
      mesh=mesh,
      compiler_params=pltpu.CompilerParams(
          use_tc_tiling_on_sc=False, needs_layout_passes=False),
      out_type=jax.ShapeDtypeStruct((L, FB, NBB, 8, C), jnp.float32),
      scratch_types=[
          pltpu.VMEM((per_w, C), jnp.int32),
          pltpu.VMEM((NB, C, D), jnp.float32),
          pltpu.VMEM((NB, D, C), jnp.float32),
          [pltpu.SemaphoreType.DMA] * NB,
          [pltpu.SemaphoreType.DMA] * NB,
      ],
  )
  def body(x_hbm, table_hbm, out_hbm, idx_v, gbuf, tbuf, gsems, ssems):
    wid = lax.axis_index("s") * NC + lax.axis_index("c")
    ubase = wid * per_w
    pltpu.sync_copy(x_hbm.at[pl.ds(ubase, per_w)], idx_v)

    def start_gather(g, b):
      pltpu.async_copy(table_hbm.at[idx_v.at[g]], gbuf.at[b], gsems[b])

    def wait_gather(b):
      pltpu.make_async_copy(table_hbm.at[idx_v.at[0]], gbuf.at[b],
                            gsems[b]).wait()

    def start_store(g, b):
      u = ubase + g
      l = u // NBB
      bb = u - l * NBB
      for fb in range(FB):
        pltpu.async_copy(tbuf.at[b, pl.ds(fb * 8, 8)],
                         out_hbm.at[l, fb, bb], ssems[b])

    def wait_store(b):
      for fb in range(FB):
        pltpu.make_async_copy(tbuf.at[b, pl.ds(fb * 8, 8)],
                              out_hbm.at[0, 0, 0], ssems[b]).wait()

    for b in range(NB):
      start_gather(b, b)

    fvecs = [lax.iota(jnp.int32, LANES) + (k * LANES) for k in range(D // LANES)]

    def unit(g, b):
      wait_gather(b)

      @pl.when(g >= NB)
      def _():
        wait_store(b)

      @plsc.parallel_loop(0, C, unroll=4)
      def _(i):
        col = jnp.full((LANES,), i, jnp.int32)
        for k in range(D // LANES):
          v = gbuf[b, i, pl.ds(k * LANES, LANES)]
          plsc.store_scatter(tbuf.at[b], [fvecs[k], col], v * SCALE)

      start_store(g, b)

      @pl.when(g + NB < per_w)
      def _():
        start_gather(g + NB, b)

    def outer(t, carry):
      for b in range(NB):
        unit(t + b, b)
      return carry

    lax.fori_loop(0, per_w // NB, lambda t, c: outer(t * NB, c), 0)

    for b in range(NB):
      wait_store(b)

  return body


def kernel(x, table):
  B, L = x.shape
  n = B * L
  nbb = B // C
  xu = x.astype(jnp.int32).T.reshape(L * nbb, C)
  out5 = _make_kernel(L, nbb)(xu, table)
  return out5.transpose(2, 4, 0, 1, 3).reshape(B, L, D)


# speedup vs baseline: 2.5662x; 1.7351x over previous
import functools
import jax
import jax.numpy as jnp
from jax import lax
from jax.experimental import pallas as pl
from jax.experimental.pallas import tpu as pltpu
from jax.experimental.pallas import tpu_sc as plsc

D = 64
SCALE = 8.0
NC = 2
NS = 16
NW = NC * NS
LANES = 16
C = 128
NB = 4
FB = D // 8


def _make_kernel(L: int, NBB: int):
  units = L * NBB
  per_w = units // NW
  assert units % NW == 0 and per_w % NB == 0

  mesh = plsc.VectorSubcoreMesh(core_axis_name="c", subcore_axis_name="s")

  @functools.partial(
      pl.kernel,
      mesh=mesh,
      compiler_params=pltpu.CompilerParams(
          use_tc_tiling_on_sc=False, needs_layout_passes=False),
      out_type=jax.ShapeDtypeStruct((L, FB, NBB, 8, C), jnp.float32),
      scratch_types=[
          pltpu.VMEM((per_w, C), jnp.int32),
          pltpu.VMEM((NB, C, D), jnp.float32),
          pltpu.VMEM((NB, D, C + 1), jnp.float32),
          [pltpu.SemaphoreType.DMA] * NB,
          [pltpu.SemaphoreType.DMA] * NB,
      ],
  )
  def body(x_hbm, table_hbm, out_hbm, idx_v, gbuf, tbuf, gsems, ssems):
    wid = lax.axis_index("s") * NC + lax.axis_index("c")
    ubase = wid * per_w
    pltpu.sync_copy(x_hbm.at[pl.ds(ubase, per_w)], idx_v)

    def start_gather(g, b):
      pltpu.async_copy(table_hbm.at[idx_v.at[g]], gbuf.at[b], gsems[b])

    def wait_gather(b):
      pltpu.make_async_copy(table_hbm.at[idx_v.at[0]], gbuf.at[b],
                            gsems[b]).wait()

    def start_store(g, b):
      u = ubase + g
      l = u // NBB
      bb = u - l * NBB
      for fb in range(FB):
        pltpu.async_copy(tbuf.at[b, pl.ds(fb * 8, 8), pl.ds(0, C)],
                         out_hbm.at[l, fb, bb], ssems[b])

    def wait_store(b):
      for fb in range(FB):
        pltpu.make_async_copy(tbuf.at[b, pl.ds(fb * 8, 8), pl.ds(0, C)],
                              out_hbm.at[0, 0, 0], ssems[b]).wait()

    for b in range(NB):
      start_gather(b, b)

    fvecs = [lax.iota(jnp.int32, LANES) + (k * LANES) for k in range(D // LANES)]

    def unit(g, b):
      wait_gather(b)

      @pl.when(g >= NB)
      def _():
        wait_store(b)

      @plsc.parallel_loop(0, C, unroll=4)
      def _(i):
        col = jnp.full((LANES,), i, jnp.int32)
        for k in range(D // LANES):
          v = gbuf[b, i, pl.ds(k * LANES, LANES)]
          plsc.store_scatter(tbuf.at[b], [fvecs[k], col], v * SCALE)

      start_store(g, b)

      @pl.when(g + NB < per_w)
      def _():
        start_gather(g + NB, b)

    def outer(t, carry):
      for b in range(NB):
        unit(t + b, b)
      return carry

    lax.fori_loop(0, per_w // NB, lambda t, c: outer(t * NB, c), 0)

    for b in range(NB):
      wait_store(b)

  return body


def kernel(x, table):
  B, L = x.shape
  n = B * L
  nbb = B // C
  xu = x.astype(jnp.int32).T.reshape(L * nbb, C)
  out5 = _make_kernel(L, nbb)(xu, table)
  return out5.transpose(2, 4, 0, 1, 3).reshape(B, L, D)


# speedup vs baseline: 2.5844x; 1.0071x over previous
import functools
import jax
import jax.numpy as jnp
from jax import lax
from jax.experimental import pallas as pl
from jax.experimental.pallas import tpu as pltpu
from jax.experimental.pallas import tpu_sc as plsc

D = 64
SCALE = 8.0
NC = 2
NS = 16
NW = NC * NS
LANES = 16
C = 128
NB = 4
FB = D // 8


def _make_kernel(L: int, NBB: int):
  units = L * NBB
  per_w = units // NW
  assert units % NW == 0 and per_w % NB == 0

  mesh = plsc.VectorSubcoreMesh(core_axis_name="c", subcore_axis_name="s")

  @functools.partial(
      pl.kernel,
      mesh=mesh,
      compiler_params=pltpu.CompilerParams(
          use_tc_tiling_on_sc=False, needs_layout_passes=False),
      out_type=jax.ShapeDtypeStruct((L, FB, NBB, 8, C), jnp.float32),
      scratch_types=[
          pltpu.VMEM((per_w, C), jnp.int32),
          pltpu.VMEM((NB, C, D), jnp.float32),
          pltpu.VMEM((NB, D, C + 1), jnp.float32),
          [pltpu.SemaphoreType.DMA] * NB,
          [pltpu.SemaphoreType.DMA] * NB,
      ],
  )
  def body(x_hbm, table_hbm, out_hbm, idx_v, gbuf, tbuf, gsems, ssems):
    wid = lax.axis_index("s") * NC + lax.axis_index("c")
    ubase = wid * per_w
    pltpu.sync_copy(x_hbm.at[pl.ds(ubase, per_w)], idx_v)

    def start_gather(g, b):
      pltpu.async_copy(table_hbm.at[idx_v.at[g]], gbuf.at[b], gsems[b])

    def wait_gather(b):
      pltpu.make_async_copy(table_hbm.at[idx_v.at[0]], gbuf.at[b],
                            gsems[b]).wait()

    def start_store(g, b):
      u = ubase + g
      l_blk = u // (NBB * 8)
      rem = u - l_blk * (NBB * 8)
      bb = rem // 8
      l = l_blk * 8 + (rem - bb * 8)
      for fb in range(FB):
        pltpu.async_copy(tbuf.at[b, pl.ds(fb * 8, 8), pl.ds(0, C)],
                         out_hbm.at[l, fb, bb], ssems[b])

    def wait_store(b):
      for fb in range(FB):
        pltpu.make_async_copy(tbuf.at[b, pl.ds(fb * 8, 8), pl.ds(0, C)],
                              out_hbm.at[0, 0, 0], ssems[b]).wait()

    for b in range(NB):
      start_gather(b, b)

    fvecs = [lax.iota(jnp.int32, LANES) + (k * LANES) for k in range(D // LANES)]

    def unit(g, b):
      wait_gather(b)

      @pl.when(g >= NB)
      def _():
        wait_store(b)

      @plsc.parallel_loop(0, C, unroll=4)
      def _(i):
        col = jnp.full((LANES,), i, jnp.int32)
        for k in range(D // LANES):
          v = gbuf[b, i, pl.ds(k * LANES, LANES)]
          plsc.store_scatter(tbuf.at[b], [fvecs[k], col], v * SCALE)

      start_store(g, b)

      @pl.when(g + NB < per_w)
      def _():
        start_gather(g + NB, b)

    def outer(t, carry):
      for b in range(NB):
        unit(t + b, b)
      return carry

    lax.fori_loop(0, per_w // NB, lambda t, c: outer(t * NB, c), 0)

    for b in range(NB):
      wait_store(b)

  return body


def kernel(x, table):
  B, L = x.shape
  n = B * L
  nbb = B // C
  xu = (x.astype(jnp.int32).T
        .reshape(L // 8, 8, nbb, C)
        .transpose(0, 2, 1, 3)
        .reshape(L * nbb, C))
  out5 = _make_kernel(L, nbb)(xu, table)
  return out5.transpose(2, 4, 0, 1, 3).reshape(B, L, D)


# speedup vs baseline: 2.5875x; 1.0012x over previous
import functools
import jax
import jax.numpy as jnp
from jax import lax
from jax.experimental import pallas as pl
from jax.experimental.pallas import tpu as pltpu
from jax.experimental.pallas import tpu_sc as plsc

D = 64
SCALE = 8.0
NC = 2
NS = 16
NW = NC * NS
LANES = 16
C = 128
NB = 4
FB = D // 8


def _make_kernel(L: int, NBB: int):
  per_w = L
  assert NBB == NW and per_w % NB == 0

  mesh = plsc.VectorSubcoreMesh(core_axis_name="c", subcore_axis_name="s")

  @functools.partial(
      pl.kernel,
      mesh=mesh,
      compiler_params=pltpu.CompilerParams(
          use_tc_tiling_on_sc=False, needs_layout_passes=False),
      out_type=jax.ShapeDtypeStruct((L, FB, NBB, 8, C), jnp.float32),
      scratch_types=[
          pltpu.VMEM((per_w, C), jnp.int32),
          pltpu.VMEM((NB, C, D), jnp.float32),
          pltpu.VMEM((NB, D, C + 1), jnp.float32),
          [pltpu.SemaphoreType.DMA] * NB,
          [pltpu.SemaphoreType.DMA] * NB,
      ],
  )
  def body(x_hbm, table_hbm, out_hbm, idx_v, gbuf, tbuf, gsems, ssems):
    wid = lax.axis_index("s") * NC + lax.axis_index("c")
    pltpu.sync_copy(x_hbm.at[:, pl.ds(wid * C, C)], idx_v)

    def start_gather(g, b):
      pltpu.async_copy(table_hbm.at[idx_v.at[g]], gbuf.at[b], gsems[b])

    def wait_gather(b):
      pltpu.make_async_copy(table_hbm.at[idx_v.at[0]], gbuf.at[b],
                            gsems[b]).wait()

    def start_store(g, b):
      for fb in range(FB):
        pltpu.async_copy(tbuf.at[b, pl.ds(fb * 8, 8), pl.ds(0, C)],
                         out_hbm.at[g, fb, wid], ssems[b])

    def wait_store(b):
      for fb in range(FB):
        pltpu.make_async_copy(tbuf.at[b, pl.ds(fb * 8, 8), pl.ds(0, C)],
                              out_hbm.at[0, 0, 0], ssems[b]).wait()

    for b in range(NB):
      start_gather(b, b)

    fvecs = [lax.iota(jnp.int32, LANES) + (k * LANES) for k in range(D // LANES)]

    def unit(g, b):
      wait_gather(b)

      @pl.when(g >= NB)
      def _():
        wait_store(b)

      @plsc.parallel_loop(0, C, unroll=4)
      def _(i):
        col = jnp.full((LANES,), i, jnp.int32)
        for k in range(D // LANES):
          v = gbuf[b, i, pl.ds(k * LANES, LANES)]
          plsc.store_scatter(tbuf.at[b], [fvecs[k], col], v * SCALE)

      start_store(g, b)

      @pl.when(g + NB < per_w)
      def _():
        start_gather(g + NB, b)

    def outer(t, carry):
      for b in range(NB):
        unit(t + b, b)
      return carry

    lax.fori_loop(0, per_w // NB, lambda t, c: outer(t * NB, c), 0)

    for b in range(NB):
      wait_store(b)

  return body


def kernel(x, table):
  B, L = x.shape
  n = B * L
  nbb = B // C
  xu = x.astype(jnp.int32).T
  out5 = _make_kernel(L, nbb)(xu, table)
  return out5.transpose(2, 4, 0, 1, 3).reshape(B, L, D)
